# baseline (speedup 1.0000x reference)
import functools

import jax
import jax.numpy as jnp
from jax import lax
from jax.experimental import pallas as pl
from jax.experimental.pallas import tpu as pltpu
from jax.experimental.pallas import tpu_sc as plsc

N = 10000
E = 320000
D = 128
DE = 144

NC = 2
NS = 16
NW = NC * NS
NP = 10240
EP = 327680
EDGES_PER_TILE = EP // NW
CHUNK = 128
CHUNKS_PER_TILE = EDGES_PER_TILE // CHUNK
ROWS_PER_TILE = NP // NS


def _sc_aggregate(feat_ext, edges):
    mesh = plsc.VectorSubcoreMesh(
        core_axis_name="c", subcore_axis_name="s", num_cores=NC,
        num_subcores=NS)

    @functools.partial(
        pl.kernel,
        out_type=jax.ShapeDtypeStruct((NC, NP, DE), jnp.float32),
        mesh=mesh,
        scratch_types=[
            pltpu.VMEM((2, CHUNK), jnp.int32),
            pltpu.VMEM((2, CHUNK), jnp.int32),
            pltpu.VMEM((CHUNK, DE), jnp.float32),
            pltpu.VMEM((CHUNK, DE), jnp.float32),
            pltpu.VMEM_SHARED((NP, DE), jnp.float32),
            pltpu.SemaphoreType.DMA,
            pltpu.SemaphoreType.DMA,
            pltpu.SemaphoreType.DMA,
            pltpu.SemaphoreType.DMA,
        ],
        compiler_params=pltpu.CompilerParams(use_tc_tiling_on_sc=False),
    )
    def agg_kernel(feat_hbm, edges_hbm, out_hbm,
                   idx_a, idx_b, rows_a, rows_b, acc_sh,
                   sem_ia, sem_ib, sem_ga, sem_gb):
        c = lax.axis_index("c")
        s = lax.axis_index("s")
        wid = c * NS + s
        row0 = s * ROWS_PER_TILE

        def wait_idx(buf, sem):
            pltpu.make_async_copy(edges_hbm.at[wid, 0], buf, sem).wait()

        def wait_rows(buf, sem):
            pltpu.make_async_copy(feat_hbm.at[idx_a.at[0]], buf, sem).wait()

        pltpu.async_copy(edges_hbm.at[wid, 0], idx_a, sem_ia)
        pltpu.async_copy(edges_hbm.at[wid, 1], idx_b, sem_ib)
        pltpu.sync_copy(feat_hbm.at[pl.ds(row0, ROWS_PER_TILE)],
                        acc_sh.at[pl.ds(row0, ROWS_PER_TILE)])
        wait_idx(idx_a, sem_ia)
        pltpu.async_copy(feat_hbm.at[idx_a.at[0]], rows_a, sem_ga)
        wait_idx(idx_b, sem_ib)
        pltpu.async_copy(feat_hbm.at[idx_b.at[0]], rows_b, sem_gb)
        plsc.subcore_barrier()

        def pair_body(g, carry):
            ca = 2 * g
            wait_rows(rows_a, sem_ga)
            pltpu.sync_copy(rows_a, acc_sh.at[idx_a.at[1]], add=True)
            pltpu.async_copy(edges_hbm.at[wid, ca + 2], idx_a, sem_ia)
            wait_rows(rows_b, sem_gb)
            pltpu.sync_copy(rows_b, acc_sh.at[idx_b.at[1]], add=True)
            pltpu.async_copy(edges_hbm.at[wid, ca + 3], idx_b, sem_ib)
            wait_idx(idx_a, sem_ia)
            pltpu.async_copy(feat_hbm.at[idx_a.at[0]], rows_a, sem_ga)
            wait_idx(idx_b, sem_ib)
            pltpu.async_copy(feat_hbm.at[idx_b.at[0]], rows_b, sem_gb)
            return carry

        lax.fori_loop(0, CHUNKS_PER_TILE // 2 - 1, pair_body, 0)
        wait_rows(rows_a, sem_ga)
        pltpu.sync_copy(rows_a, acc_sh.at[idx_a.at[1]], add=True)
        wait_rows(rows_b, sem_gb)
        pltpu.sync_copy(rows_b, acc_sh.at[idx_b.at[1]], add=True)
        plsc.subcore_barrier()
        pltpu.sync_copy(acc_sh.at[pl.ds(row0, ROWS_PER_TILE)],
                        out_hbm.at[c, pl.ds(row0, ROWS_PER_TILE)])

    return agg_kernel(feat_ext, edges)


def _tc_body(agg_ref, feat_ref, w_ref, out_ref):
    a = agg_ref[0] + agg_ref[1]
    num = a[:, :D] - feat_ref[...]
    den = a[:, D:D + 1] - 1.0
    h = num / den
    out_ref[...] = jnp.maximum(
        jnp.dot(h, w_ref[...], preferred_element_type=jnp.float32), 0.0)


def _tc_finalize(agg, feat, w):
    br = 400
    return pl.pallas_call(
        _tc_body,
        out_shape=jax.ShapeDtypeStruct((N, D), jnp.float32),
        grid=(N // br,),
        in_specs=[
            pl.BlockSpec((NC, br, DE), lambda i: (0, i, 0)),
            pl.BlockSpec((br, D), lambda i: (i, 0)),
            pl.BlockSpec((D, D), lambda i: (0, 0)),
        ],
        out_specs=pl.BlockSpec((br, D), lambda i: (i, 0)),
    )(agg, feat, w)


def kernel(feat, edge_index, W):
    feat_ext = jnp.concatenate(
        [jnp.pad(feat, ((0, NP - N), (0, 0))),
         jnp.ones((NP, 1), dtype=jnp.float32),
         jnp.zeros((NP, DE - D - 1), dtype=jnp.float32)], axis=1)
    pad = jnp.concatenate(
        [jnp.zeros((1, EP - E), jnp.int32),
         N + (jnp.arange(EP - E, dtype=jnp.int32) % (NP - N))[None]],
        axis=0)
    ep = jnp.concatenate([edge_index, pad], axis=1)
    edges = jnp.stack(
        [ep[0].reshape(NW, CHUNKS_PER_TILE, CHUNK),
         ep[1].reshape(NW, CHUNKS_PER_TILE, CHUNK)], axis=2)
    agg = _sc_aggregate(feat_ext, edges)
    return _tc_finalize(agg, feat, W)


# speedup vs baseline: 1.9319x; 1.9319x over previous
import functools

import jax
import jax.numpy as jnp
from jax import lax
from jax.experimental import pallas as pl
from jax.experimental.pallas import tpu as pltpu
from jax.experimental.pallas import tpu_sc as plsc

N = 10000
E = 320000
D = 128
DE = 144

NC = 2
NS = 16
NW = NC * NS
NP = 10240
EP = E
EDGES_PER_TILE = EP // NW
CHUNK = 100
CHUNKS_PER_TILE = EDGES_PER_TILE // CHUNK
ROWS_PER_TILE = NP // NS


def _sc_aggregate(feat_ext, edges):
    mesh = plsc.VectorSubcoreMesh(
        core_axis_name="c", subcore_axis_name="s", num_cores=NC,
        num_subcores=NS)

    @functools.partial(
        pl.kernel,
        out_type=jax.ShapeDtypeStruct((NC, NP, DE), jnp.float32),
        mesh=mesh,
        scratch_types=[
            pltpu.VMEM((2, CHUNK), jnp.int32),
            pltpu.VMEM((2, CHUNK), jnp.int32),
            pltpu.VMEM((CHUNK, DE), jnp.float32),
            pltpu.VMEM((CHUNK, DE), jnp.float32),
            pltpu.VMEM_SHARED((NP, DE), jnp.float32),
            pltpu.SemaphoreType.DMA,
            pltpu.SemaphoreType.DMA,
            pltpu.SemaphoreType.DMA,
            pltpu.SemaphoreType.DMA,
            pltpu.SemaphoreType.DMA,
            pltpu.SemaphoreType.DMA,
        ],
        compiler_params=pltpu.CompilerParams(use_tc_tiling_on_sc=False),
    )
    def agg_kernel(featext_hbm, edges_hbm, out_hbm,
                   idx_a, idx_b, rows_a, rows_b, acc_sh,
                   sem_ia, sem_ib, sem_ga, sem_gb, sem_sa, sem_sb):
        c = lax.axis_index("c")
        s = lax.axis_index("s")
        wid = c * NS + s
        row0 = s * ROWS_PER_TILE

        def wait_idx(buf, sem):
            pltpu.make_async_copy(edges_hbm.at[wid, 0], buf, sem).wait()

        def gather(buf, idx, sem):
            pltpu.async_copy(featext_hbm.at[idx.at[0]], buf, sem)

        def wait_rows(buf, sem):
            pltpu.make_async_copy(featext_hbm.at[idx_a.at[0]],
                                  buf, sem).wait()

        def scatter(buf, idx, sem):
            pltpu.async_copy(buf, acc_sh.at[idx.at[1]], sem, add=True)

        def wait_scatter(buf, sem):
            pltpu.make_async_copy(buf, acc_sh.at[idx_a.at[1]], sem).wait()

        pltpu.async_copy(edges_hbm.at[wid, 0], idx_a, sem_ia)
        pltpu.async_copy(edges_hbm.at[wid, 1], idx_b, sem_ib)
        pltpu.sync_copy(featext_hbm.at[pl.ds(row0, ROWS_PER_TILE)],
                        acc_sh.at[pl.ds(row0, ROWS_PER_TILE)])
        wait_idx(idx_a, sem_ia)
        gather(rows_a, idx_a, sem_ga)
        wait_idx(idx_b, sem_ib)
        gather(rows_b, idx_b, sem_gb)
        plsc.subcore_barrier()

        def pair_body(g, carry):
            ca = 2 * g
            wait_rows(rows_a, sem_ga)
            scatter(rows_a, idx_a, sem_sa)
            wait_rows(rows_b, sem_gb)
            scatter(rows_b, idx_b, sem_sb)
            wait_scatter(rows_a, sem_sa)
            pltpu.async_copy(edges_hbm.at[wid, ca + 2], idx_a, sem_ia)
            wait_scatter(rows_b, sem_sb)
            pltpu.async_copy(edges_hbm.at[wid, ca + 3], idx_b, sem_ib)
            wait_idx(idx_a, sem_ia)
            gather(rows_a, idx_a, sem_ga)
            wait_idx(idx_b, sem_ib)
            gather(rows_b, idx_b, sem_gb)
            return carry

        lax.fori_loop(0, CHUNKS_PER_TILE // 2 - 1, pair_body, 0)
        wait_rows(rows_a, sem_ga)
        scatter(rows_a, idx_a, sem_sa)
        wait_rows(rows_b, sem_gb)
        scatter(rows_b, idx_b, sem_sb)
        wait_scatter(rows_a, sem_sa)
        wait_scatter(rows_b, sem_sb)
        plsc.subcore_barrier()
        pltpu.sync_copy(acc_sh.at[pl.ds(row0, ROWS_PER_TILE)],
                        out_hbm.at[c, pl.ds(row0, ROWS_PER_TILE)])

    return agg_kernel(feat_ext, edges)


def _tc_body(agg_ref, feat_ref, w_ref, out_ref):
    a = agg_ref[0] + agg_ref[1]
    num = a[:, :D] - feat_ref[...]
    den = a[:, D:D + 1] - 1.0
    h = num / den
    out_ref[...] = jnp.maximum(
        jnp.dot(h, w_ref[...], preferred_element_type=jnp.float32), 0.0)


def _tc_finalize(agg, feat, w):
    br = 400
    return pl.pallas_call(
        _tc_body,
        out_shape=jax.ShapeDtypeStruct((N, D), jnp.float32),
        grid=(N // br,),
        in_specs=[
            pl.BlockSpec((NC, br, DE), lambda i: (0, i, 0)),
            pl.BlockSpec((br, D), lambda i: (i, 0)),
            pl.BlockSpec((D, D), lambda i: (0, 0)),
        ],
        out_specs=pl.BlockSpec((br, D), lambda i: (i, 0)),
    )(agg, feat, w)


def kernel(feat, edge_index, W):
    feat_ext = jnp.concatenate(
        [jnp.pad(feat, ((0, NP - N), (0, 0))),
         jnp.ones((NP, 1), dtype=jnp.float32),
         jnp.zeros((NP, DE - D - 1), dtype=jnp.float32)], axis=1)
    pad = jnp.concatenate(
        [jnp.zeros((1, EP - E), jnp.int32),
         N + (jnp.arange(EP - E, dtype=jnp.int32) % (NP - N))[None]],
        axis=0)
    ep = jnp.concatenate([edge_index, pad], axis=1)
    edges = jnp.stack(
        [ep[0].reshape(NW, CHUNKS_PER_TILE, CHUNK),
         ep[1].reshape(NW, CHUNKS_PER_TILE, CHUNK)], axis=2)
    agg = _sc_aggregate(feat_ext, edges)
    return _tc_finalize(agg, feat, W)


# speedup vs baseline: 2.1605x; 1.1184x over previous
import functools

import jax
import jax.numpy as jnp
from jax import lax
from jax.experimental import pallas as pl
from jax.experimental.pallas import tpu as pltpu
from jax.experimental.pallas import tpu_sc as plsc

N = 10000
E = 320000
D = 128

NC = 2
NS = 16
NW = NC * NS
NP = 10240
EDGES_PER_TILE = E // NW
CHUNK = 100
CHUNKS_PER_TILE = EDGES_PER_TILE // CHUNK
ROWS_PER_TILE = NP // NS
DW = 16


def _sc_aggregate(feat, edges, zeros):
    mesh = plsc.VectorSubcoreMesh(
        core_axis_name="c", subcore_axis_name="s", num_cores=NC,
        num_subcores=NS)

    @functools.partial(
        pl.kernel,
        out_type=(jax.ShapeDtypeStruct((NC, NP, D), jnp.float32),
                  jax.ShapeDtypeStruct((NC, NP, DW), jnp.float32)),
        mesh=mesh,
        scratch_types=[
            pltpu.VMEM((2, CHUNK), jnp.int32),
            pltpu.VMEM((2, CHUNK), jnp.int32),
            pltpu.VMEM((CHUNK, D), jnp.float32),
            pltpu.VMEM((CHUNK, D), jnp.float32),
            pltpu.VMEM((CHUNK, DW), jnp.float32),
            pltpu.VMEM_SHARED((NP, D), jnp.float32),
            pltpu.VMEM_SHARED((NP, DW), jnp.float32),
            pltpu.SemaphoreType.DMA,
            pltpu.SemaphoreType.DMA,
            pltpu.SemaphoreType.DMA,
            pltpu.SemaphoreType.DMA,
        ],
        compiler_params=pltpu.CompilerParams(use_tc_tiling_on_sc=False),
    )
    def agg_kernel(feat_hbm, edges_hbm, zeros_hbm, acc_out, deg_out,
                   idx_a, idx_b, rows_a, rows_b, ones_v,
                   acc_sh, deg_sh, sem_ia, sem_ib, sem_ga, sem_gb):
        c = lax.axis_index("c")
        s = lax.axis_index("s")
        wid = c * NS + s
        row0 = s * ROWS_PER_TILE

        def wait_idx(buf, sem):
            pltpu.make_async_copy(edges_hbm.at[wid, 0], buf, sem).wait()

        def gather(buf, idx, sem):
            pltpu.async_copy(feat_hbm.at[idx.at[0]], buf, sem)

        def wait_rows(buf, sem):
            pltpu.make_async_copy(feat_hbm.at[idx_a.at[0]], buf, sem).wait()

        def count_degrees(idx):
            pltpu.sync_copy(ones_v, deg_sh.at[idx.at[1]], add=True)

        pltpu.async_copy(edges_hbm.at[wid, 0], idx_a, sem_ia)
        pltpu.async_copy(edges_hbm.at[wid, 1], idx_b, sem_ib)
        ones = jnp.ones((DW,), jnp.float32)
        def fill_ones(i, carry):
            ones_v[i] = ones
            return carry
        lax.fori_loop(0, CHUNK, fill_ones, 0)
        pltpu.sync_copy(feat_hbm.at[pl.ds(row0, ROWS_PER_TILE)],
                        acc_sh.at[pl.ds(row0, ROWS_PER_TILE)])
        pltpu.sync_copy(zeros_hbm.at[pl.ds(row0, ROWS_PER_TILE)],
                        deg_sh.at[pl.ds(row0, ROWS_PER_TILE)])
        wait_idx(idx_a, sem_ia)
        gather(rows_a, idx_a, sem_ga)
        wait_idx(idx_b, sem_ib)
        gather(rows_b, idx_b, sem_gb)
        plsc.subcore_barrier()

        def pair_body(g, carry):
            ca = 2 * g
            wait_rows(rows_a, sem_ga)
            pltpu.sync_copy(rows_a, acc_sh.at[idx_a.at[1]], add=True)
            count_degrees(idx_a)
            pltpu.async_copy(edges_hbm.at[wid, ca + 2], idx_a, sem_ia)
            wait_rows(rows_b, sem_gb)
            pltpu.sync_copy(rows_b, acc_sh.at[idx_b.at[1]], add=True)
            count_degrees(idx_b)
            pltpu.async_copy(edges_hbm.at[wid, ca + 3], idx_b, sem_ib)
            wait_idx(idx_a, sem_ia)
            gather(rows_a, idx_a, sem_ga)
            wait_idx(idx_b, sem_ib)
            gather(rows_b, idx_b, sem_gb)
            return carry

        lax.fori_loop(0, CHUNKS_PER_TILE // 2 - 1, pair_body, 0)
        wait_rows(rows_a, sem_ga)
        pltpu.sync_copy(rows_a, acc_sh.at[idx_a.at[1]], add=True)
        count_degrees(idx_a)
        wait_rows(rows_b, sem_gb)
        pltpu.sync_copy(rows_b, acc_sh.at[idx_b.at[1]], add=True)
        count_degrees(idx_b)
        plsc.subcore_barrier()
        pltpu.sync_copy(acc_sh.at[pl.ds(row0, ROWS_PER_TILE)],
                        acc_out.at[c, pl.ds(row0, ROWS_PER_TILE)])
        pltpu.sync_copy(deg_sh.at[pl.ds(row0, ROWS_PER_TILE)],
                        deg_out.at[c, pl.ds(row0, ROWS_PER_TILE)])

    return agg_kernel(feat, edges, zeros)


def _tc_body(agg_ref, deg_ref, feat_ref, w_ref, out_ref):
    a = agg_ref[0] + agg_ref[1] - feat_ref[...]
    m = jnp.dot(a, w_ref[...], preferred_element_type=jnp.float32)
    d = deg_ref[0, :, 0] + deg_ref[1, :, 0] + 1.0
    dcol = jnp.broadcast_to(d, (D, D)).T
    out_ref[...] = jnp.maximum(m / dcol, 0.0)


def _tc_finalize(agg, deg, feat, w):
    br = 128
    return pl.pallas_call(
        _tc_body,
        out_shape=jax.ShapeDtypeStruct((N, D), jnp.float32),
        grid=(pl.cdiv(N, br),),
        in_specs=[
            pl.BlockSpec((NC, br, D), lambda i: (0, i, 0)),
            pl.BlockSpec((NC, br, DW), lambda i: (0, i, 0)),
            pl.BlockSpec((br, D), lambda i: (i, 0)),
            pl.BlockSpec((D, D), lambda i: (0, 0)),
        ],
        out_specs=pl.BlockSpec((br, D), lambda i: (i, 0)),
    )(agg, deg, feat, w)


def kernel(feat, edge_index, W):
    feat_pad = jnp.pad(feat, ((0, NP - N), (0, 0)))
    edges = jnp.stack(
        [edge_index[0].reshape(NW, CHUNKS_PER_TILE, CHUNK),
         edge_index[1].reshape(NW, CHUNKS_PER_TILE, CHUNK)], axis=2)
    zeros = jnp.zeros((NP, DW), jnp.float32)
    agg, deg = _sc_aggregate(feat_pad, edges, zeros)
    return _tc_finalize(agg, deg, feat, W)


# speedup vs baseline: 2.1700x; 1.0044x over previous
import functools

import jax
import jax.numpy as jnp
from jax import lax
from jax.experimental import pallas as pl
from jax.experimental.pallas import tpu as pltpu
from jax.experimental.pallas import tpu_sc as plsc

N = 10000
E = 320000
D = 128
DE = 144

NC = 2
NS = 16
NW = NC * NS
NP = 10240
EDGES_PER_TILE = E // NW
CHUNK = 100
CHUNKS_PER_TILE = EDGES_PER_TILE // CHUNK
ROWS_PER_TILE = NP // NS


def _sc_aggregate(feat_ext, edges):
    mesh = plsc.VectorSubcoreMesh(
        core_axis_name="c", subcore_axis_name="s", num_cores=NC,
        num_subcores=NS)

    @functools.partial(
        pl.kernel,
        out_type=jax.ShapeDtypeStruct((NC, NP, DE), jnp.float32),
        mesh=mesh,
        scratch_types=[
            pltpu.VMEM((2, CHUNK), jnp.int32),
            pltpu.VMEM((2, CHUNK), jnp.int32),
            pltpu.VMEM((CHUNK, DE), jnp.float32),
            pltpu.VMEM((CHUNK, DE), jnp.float32),
            pltpu.VMEM_SHARED((NP, DE), jnp.float32),
            pltpu.SemaphoreType.DMA,
            pltpu.SemaphoreType.DMA,
            pltpu.SemaphoreType.DMA,
            pltpu.SemaphoreType.DMA,
        ],
        compiler_params=pltpu.CompilerParams(use_tc_tiling_on_sc=False),
    )
    def agg_kernel(feat_hbm, edges_hbm, out_hbm,
                   idx_a, idx_b, rows_a, rows_b, acc_sh,
                   sem_ia, sem_ib, sem_ga, sem_gb):
        c = lax.axis_index("c")
        s = lax.axis_index("s")
        wid = c * NS + s
        row0 = s * ROWS_PER_TILE

        def wait_idx(buf, sem):
            pltpu.make_async_copy(edges_hbm.at[wid, 0], buf, sem).wait()

        def wait_rows(buf, sem):
            pltpu.make_async_copy(feat_hbm.at[idx_a.at[0]], buf, sem).wait()

        pltpu.async_copy(edges_hbm.at[wid, 0], idx_a, sem_ia)
        pltpu.async_copy(edges_hbm.at[wid, 1], idx_b, sem_ib)
        pltpu.sync_copy(feat_hbm.at[pl.ds(row0, ROWS_PER_TILE)],
                        acc_sh.at[pl.ds(row0, ROWS_PER_TILE)])
        wait_idx(idx_a, sem_ia)
        pltpu.async_copy(feat_hbm.at[idx_a.at[0]], rows_a, sem_ga)
        wait_idx(idx_b, sem_ib)
        pltpu.async_copy(feat_hbm.at[idx_b.at[0]], rows_b, sem_gb)
        plsc.subcore_barrier()

        def pair_body(g, carry):
            ca = 2 * g
            wait_rows(rows_a, sem_ga)
            pltpu.sync_copy(rows_a, acc_sh.at[idx_a.at[1]], add=True)
            pltpu.async_copy(edges_hbm.at[wid, ca + 2], idx_a, sem_ia)
            wait_rows(rows_b, sem_gb)
            pltpu.sync_copy(rows_b, acc_sh.at[idx_b.at[1]], add=True)
            pltpu.async_copy(edges_hbm.at[wid, ca + 3], idx_b, sem_ib)
            wait_idx(idx_a, sem_ia)
            pltpu.async_copy(feat_hbm.at[idx_a.at[0]], rows_a, sem_ga)
            wait_idx(idx_b, sem_ib)
            pltpu.async_copy(feat_hbm.at[idx_b.at[0]], rows_b, sem_gb)
            return carry

        lax.fori_loop(0, CHUNKS_PER_TILE // 2 - 1, pair_body, 0)
        wait_rows(rows_a, sem_ga)
        pltpu.sync_copy(rows_a, acc_sh.at[idx_a.at[1]], add=True)
        wait_rows(rows_b, sem_gb)
        pltpu.sync_copy(rows_b, acc_sh.at[idx_b.at[1]], add=True)
        plsc.subcore_barrier()
        pltpu.sync_copy(acc_sh.at[pl.ds(row0, ROWS_PER_TILE)],
                        out_hbm.at[c, pl.ds(row0, ROWS_PER_TILE)])

    return agg_kernel(feat_ext, edges)


def _tc_body(agg_ref, feat_ref, w_ref, out_ref):
    a = agg_ref[0] + agg_ref[1]
    num = a[:, :D] - feat_ref[...]
    den = a[:, D:D + 1] - 1.0
    h = num / den
    out_ref[...] = jnp.maximum(
        jnp.dot(h, w_ref[...], preferred_element_type=jnp.float32), 0.0)


def _tc_finalize(agg, feat, w):
    br = 400
    return pl.pallas_call(
        _tc_body,
        out_shape=jax.ShapeDtypeStruct((N, D), jnp.float32),
        grid=(N // br,),
        in_specs=[
            pl.BlockSpec((NC, br, DE), lambda i: (0, i, 0)),
            pl.BlockSpec((br, D), lambda i: (i, 0)),
            pl.BlockSpec((D, D), lambda i: (0, 0)),
        ],
        out_specs=pl.BlockSpec((br, D), lambda i: (i, 0)),
    )(agg, feat, w)


def kernel(feat, edge_index, W):
    feat_ext = jnp.concatenate(
        [jnp.pad(feat, ((0, NP - N), (0, 0))),
         jnp.ones((NP, 1), dtype=jnp.float32),
         jnp.zeros((NP, DE - D - 1), dtype=jnp.float32)], axis=1)
    edges = jnp.stack(
        [edge_index[0].reshape(NW, CHUNKS_PER_TILE, CHUNK),
         edge_index[1].reshape(NW, CHUNKS_PER_TILE, CHUNK)], axis=2)
    agg = _sc_aggregate(feat_ext, edges)
    return _tc_finalize(agg, feat, W)
